# baseline (speedup 1.0000x reference)
import functools

import jax
import jax.numpy as jnp
from jax import lax
from jax.experimental import pallas as pl
from jax.experimental.pallas import tpu as pltpu
from jax.experimental.pallas import tpu_sc as plsc

STREAM = 128


@functools.lru_cache(maxsize=None)
def _make_gather(B, D):
    info = plsc.get_sparse_core_info()
    NC, NS = info.num_cores, info.num_subcores
    NW = NC * NS
    assert B % (NW * STREAM) == 0
    per_w = B // NW
    n_streams = per_w // STREAM
    mesh = plsc.VectorSubcoreMesh(core_axis_name="c", subcore_axis_name="s")

    @functools.partial(
        pl.kernel,
        mesh=mesh,
        compiler_params=pltpu.CompilerParams(use_tc_tiling_on_sc=False),
        out_type=jax.ShapeDtypeStruct((B, D), jnp.float32),
        scratch_types=[
            pltpu.VMEM((n_streams, STREAM), jnp.int32),
            pltpu.VMEM((STREAM, D), jnp.float32),
            pltpu.SemaphoreType.DMA,
        ],
    )
    def k(nodes_hbm, table_hbm, out_hbm, idx_v, rows_v, sem):
        wid = lax.axis_index("s") * NC + lax.axis_index("c")
        base = wid * per_w
        pltpu.sync_copy(nodes_hbm.at[pl.ds(wid * n_streams, n_streams)], idx_v)

        def body(j, carry):
            pltpu.async_copy(table_hbm.at[idx_v.at[j]], rows_v, sem).wait()
            pltpu.sync_copy(rows_v, out_hbm.at[pl.ds(base + j * STREAM, STREAM)])
            return carry

        lax.fori_loop(0, n_streams, body, 0)

    return k


def kernel(nodes, table):
    n, w = nodes.shape
    B = n * w
    idx2d = nodes.reshape(B // STREAM, STREAM).astype(jnp.int32)
    out = _make_gather(B, table.shape[1])(idx2d, table)
    return out.reshape(n, w, table.shape[1])


# speedup vs baseline: 1.0584x; 1.0584x over previous
import functools

import jax
import jax.numpy as jnp
from jax import lax
from jax.experimental import pallas as pl
from jax.experimental.pallas import tpu as pltpu
from jax.experimental.pallas import tpu_sc as plsc

STREAM = 128
CHUNK = 5
CROWS = CHUNK * STREAM


@functools.lru_cache(maxsize=None)
def _make_gather(B, D):
    info = plsc.get_sparse_core_info()
    NC, NS = info.num_cores, info.num_subcores
    NW = NC * NS
    assert B % (NW * STREAM * CHUNK * 2) == 0
    per_w = B // NW
    n_streams = per_w // STREAM
    n_chunks = n_streams // CHUNK
    mesh = plsc.VectorSubcoreMesh(core_axis_name="c", subcore_axis_name="s")

    @functools.partial(
        pl.kernel,
        mesh=mesh,
        compiler_params=pltpu.CompilerParams(use_tc_tiling_on_sc=False),
        out_type=jax.ShapeDtypeStruct((B, D), jnp.float32),
        scratch_types=[
            pltpu.VMEM((n_streams, STREAM), jnp.int32),
            pltpu.VMEM((CROWS, D), jnp.float32),
            pltpu.VMEM((CROWS, D), jnp.float32),
            pltpu.SemaphoreType.DMA,
            pltpu.SemaphoreType.DMA,
            pltpu.SemaphoreType.DMA,
            pltpu.SemaphoreType.DMA,
        ],
    )
    def k(nodes_hbm, table_hbm, out_hbm, idx_v, buf0, buf1,
          gsem0, gsem1, wsem0, wsem1):
        wid = lax.axis_index("s") * NC + lax.axis_index("c")
        base = wid * per_w
        pltpu.sync_copy(nodes_hbm.at[pl.ds(wid * n_streams, n_streams)], idx_v)

        def body(t, carry):
            e = 2 * t
            o = e + 1
            gh0 = [pltpu.async_copy(table_hbm.at[idx_v.at[e * CHUNK + s]],
                                    buf0.at[pl.ds(s * STREAM, STREAM)], gsem0)
                   for s in range(CHUNK)]
            gh1 = [pltpu.async_copy(table_hbm.at[idx_v.at[o * CHUNK + s]],
                                    buf1.at[pl.ds(s * STREAM, STREAM)], gsem1)
                   for s in range(CHUNK)]
            for h in gh0:
                h.wait()
            w0 = pltpu.async_copy(buf0, out_hbm.at[pl.ds(base + e * CROWS, CROWS)],
                                  wsem0)
            for h in gh1:
                h.wait()
            w1 = pltpu.async_copy(buf1, out_hbm.at[pl.ds(base + o * CROWS, CROWS)],
                                  wsem1)
            w0.wait()
            w1.wait()
            return carry

        lax.fori_loop(0, n_chunks // 2, body, 0)

    return k


def kernel(nodes, table):
    n, w = nodes.shape
    B = n * w
    idx2d = nodes.reshape(B // STREAM, STREAM).astype(jnp.int32)
    out = _make_gather(B, table.shape[1])(idx2d, table)
    return out.reshape(n, w, table.shape[1])
